# baseline (speedup 1.0000x reference)
import jax
import jax.numpy as jnp
from jax import lax
from jax.experimental import pallas as pl
from jax.experimental.pallas import tpu as pltpu
from jax.experimental.pallas import tpu_sc as plsc

EMBED = 64
BATCH = 16384
NC = 2
NS = 16
L = 16
NW = NC * NS
BPW = BATCH // NW
CHUNK = 128
NCHUNK = BPW // CHUNK


def _sc_body(u_hbm, v_hbm, ut_hbm, it_hbm, out_hbm,
             uidx, vidx, urows, vrows, sbuf, outv, sem):
    wid = lax.axis_index("s") * NC + lax.axis_index("c")
    base = wid * BPW

    pltpu.sync_copy(u_hbm.at[wid], uidx)
    pltpu.sync_copy(v_hbm.at[wid], vidx)

    copies = []
    for j in range(NCHUNK):
        copies.append(pltpu.async_copy(
            ut_hbm.at[uidx.at[j]], urows.at[pl.ds(j * CHUNK, CHUNK)], sem))
        copies.append(pltpu.async_copy(
            it_hbm.at[vidx.at[j]], vrows.at[pl.ds(j * CHUNK, CHUNK)], sem))
    for c in copies:
        c.wait()

    lanes = lax.iota(jnp.int32, L)

    def group(g, carry):
        tot = jnp.zeros((L,), jnp.float32)
        for r in range(L):
            row = g * L + r
            acc = urows[row, pl.ds(0, L)] * vrows[row, pl.ds(0, L)]
            for e in range(1, EMBED // L):
                acc = acc + urows[row, pl.ds(e * L, L)] * vrows[row, pl.ds(e * L, L)]
            tot = jnp.where(lanes == r, jnp.sum(acc), tot)
        outv[pl.ds(g * L, L)] = tot
        return carry

    lax.fori_loop(0, BPW // L, group, 0)
    pltpu.sync_copy(outv, out_hbm.at[pl.ds(base, BPW)])


def kernel(u, v, user_table, item_table):
    u3 = u.astype(jnp.int32).reshape(NW, NCHUNK, CHUNK)
    v3 = v.astype(jnp.int32).reshape(NW, NCHUNK, CHUNK)
    mesh = plsc.VectorSubcoreMesh(core_axis_name="c", subcore_axis_name="s")
    f = pl.kernel(
        _sc_body,
        out_type=jax.ShapeDtypeStruct((BATCH,), jnp.float32),
        mesh=mesh,
        compiler_params=pltpu.CompilerParams(
            needs_layout_passes=False, use_tc_tiling_on_sc=False),
        scratch_types=[
            pltpu.VMEM((NCHUNK, CHUNK), jnp.int32),
            pltpu.VMEM((NCHUNK, CHUNK), jnp.int32),
            pltpu.VMEM((BPW, EMBED), jnp.float32),
            pltpu.VMEM((BPW, EMBED), jnp.float32),
            pltpu.VMEM((L * L,), jnp.float32),
            pltpu.VMEM((BPW,), jnp.float32),
            pltpu.SemaphoreType.DMA,
        ],
    )
    return f(u3, v3, user_table, item_table)


# speedup vs baseline: 1.2383x; 1.2383x over previous
import jax
import jax.numpy as jnp
from jax import lax
from jax.experimental import pallas as pl
from jax.experimental.pallas import tpu as pltpu
from jax.experimental.pallas import tpu_sc as plsc

EMBED = 64
BATCH = 16384
NC = 2
NS = 16
L = 16
NW = NC * NS
BPW = BATCH // NW
C = 32
NCH = BPW // C


def _sc_body(u_hbm, v_hbm, ut_hbm, it_hbm, out_hbm,
             uidx, vidx, ubuf, vbuf, outv, sem):
    wid = lax.axis_index("s") * NC + lax.axis_index("c")
    base = wid * BPW

    pltpu.sync_copy(u_hbm.at[wid], uidx)
    pltpu.sync_copy(v_hbm.at[wid], vidx)

    lanes = lax.iota(jnp.int32, L)

    def chunk(ci, carry):
        handles = []
        for g in range(C // L):
            uvec = uidx[pl.ds(ci * C + g * L, L)]
            vvec = vidx[pl.ds(ci * C + g * L, L)]
            for r in range(L):
                j = g * L + r
                handles.append(pltpu.async_copy(ut_hbm.at[uvec[r]], ubuf.at[j], sem))
                handles.append(pltpu.async_copy(it_hbm.at[vvec[r]], vbuf.at[j], sem))
        for h in handles:
            h.wait()
        for g in range(C // L):
            tot = jnp.zeros((L,), jnp.float32)
            for r in range(L):
                j = g * L + r
                acc = ubuf[j, pl.ds(0, L)] * vbuf[j, pl.ds(0, L)]
                for e in range(1, EMBED // L):
                    acc = acc + ubuf[j, pl.ds(e * L, L)] * vbuf[j, pl.ds(e * L, L)]
                tot = jnp.where(lanes == r, jnp.sum(acc), tot)
            outv[pl.ds(ci * C + g * L, L)] = tot
        return carry

    lax.fori_loop(0, NCH, chunk, 0)
    pltpu.sync_copy(outv, out_hbm.at[pl.ds(base, BPW)])


def kernel(u, v, user_table, item_table):
    u2 = u.astype(jnp.int32).reshape(NW, BPW)
    v2 = v.astype(jnp.int32).reshape(NW, BPW)
    mesh = plsc.VectorSubcoreMesh(core_axis_name="c", subcore_axis_name="s")
    f = pl.kernel(
        _sc_body,
        out_type=jax.ShapeDtypeStruct((BATCH,), jnp.float32),
        mesh=mesh,
        compiler_params=pltpu.CompilerParams(
            needs_layout_passes=False, use_tc_tiling_on_sc=True),
        scratch_types=[
            pltpu.VMEM((BPW,), jnp.int32),
            pltpu.VMEM((BPW,), jnp.int32),
            pltpu.VMEM((C, EMBED), jnp.float32),
            pltpu.VMEM((C, EMBED), jnp.float32),
            pltpu.VMEM((BPW,), jnp.float32),
            pltpu.SemaphoreType.DMA,
        ],
    )
    return f(u2, v2, user_table, item_table)
